# baseline (speedup 1.0000x reference)
import functools

import jax
import jax.numpy as jnp
from jax import lax
from jax.experimental import pallas as pl
from jax.experimental.pallas import tpu as pltpu
from jax.experimental.pallas import tpu_sc as plsc

Q = 1024
K = 64
N = 100000
NPAD = 102400
TILE = 4096
GRID = NPAD // TILE
NT = NPAD // 128
NTV = NT // 16
TOPK = 100
OW = 128
CAP = 448
CKW = CAP + 16
EQW = 144
NBIN = 1024
LLEN = 64
ROWS_PER_W = Q // 32
INT_MIN32 = -2147483648
PAD_BITS = -8388607


def _mm_body(a_ref, b_ref, s_ref, tm_ref):
    j = pl.program_id(0)
    s = jnp.dot(a_ref[...], b_ref[...], preferred_element_type=jnp.float32)
    col = j * TILE + lax.broadcasted_iota(jnp.int32, s.shape, 1)
    valid = col < N
    s_masked = jnp.where(valid, s, -jnp.inf)
    tm_ref[...] = jnp.max(
        s_masked.reshape(Q, TILE // 128, 128), axis=2)[None]
    pad_val = lax.bitcast_convert_type(jnp.int32(PAD_BITS), jnp.float32)
    s_ref[...] = jnp.where(valid, s, pad_val)


def _scores(unmatch_entities3, all_candidates3):
    b = jnp.pad(all_candidates3, ((0, 0), (0, NPAD - N)))
    return pl.pallas_call(
        _mm_body,
        grid=(GRID,),
        in_specs=[
            pl.BlockSpec((Q, K), lambda j: (0, 0)),
            pl.BlockSpec((K, TILE), lambda j: (0, j)),
        ],
        out_specs=[
            pl.BlockSpec((Q, TILE), lambda j: (0, j)),
            pl.BlockSpec((1, Q, TILE // 128), lambda j: (j, 0, 0)),
        ],
        out_shape=[
            jax.ShapeDtypeStruct((Q, NPAD), jnp.float32),
            jax.ShapeDtypeStruct((GRID, Q, TILE // 128), jnp.float32),
        ],
    )(unmatch_entities3, b)


def _lane():
    return lax.iota(jnp.int32, 16)


def _u_of(v):
    b = lax.bitcast_convert_type(v, jnp.int32)
    return jnp.where(b < 0, ~b, b | INT_MIN32)


def _zero_hist(hist):
    def body(i, _):
        hist[pl.ds(i * 16, 16)] = jnp.zeros((16,), jnp.int32)
        return 0
    lax.fori_loop(0, NBIN, body, 0)


def _hist_pass(gbuf, ntile, hist, masked, s_hi, pfx, s_lo, wmask):
    lane = _lane()
    ones = jnp.ones((16,), jnp.int32)
    def body(t, _):
        for j in range(8):
            u = _u_of(gbuf[t, pl.ds(j * 16, 16)])
            if masked:
                m = lax.shift_right_logical(u, s_hi) == pfx
                bin_ = lax.shift_right_logical(u, s_lo) & wmask
                plsc.addupdate_scatter(hist, [lane * NBIN + bin_], ones,
                                       mask=m)
            else:
                bin_ = lax.shift_right_logical(u, 22)
                plsc.addupdate_scatter(hist, [lane * NBIN + bin_], ones)
        return 0
    lax.fori_loop(0, ntile, body, 0)


def _reduce_hist(hist, totals):
    def body(g, _):
        acc = jnp.zeros((16,), jnp.int32)
        for l in range(16):
            acc = acc + hist[pl.ds(l * NBIN + g * 16, 16)]
        totals[pl.ds(g * 16, 16)] = acc
        return 0
    lax.fori_loop(0, NBIN // 16, body, 0)


def _scan_bins(totals, cum0):
    lane = _lane()
    def body(g, carry):
        cum, found, B, na, cnt = carry
        gi = 63 - g
        t = totals[pl.ds(gi * 16, 16)]
        tr = lax.rev(t, (0,))
        tot_ge = plsc.cumsum(tr) + cum
        posv = jnp.where(tot_ge >= TOPK, lane, 16)
        pos = jnp.min(posv)
        any_ = pos < 16
        sel = lane == pos
        tge_at = jnp.sum(jnp.where(sel, tot_ge, 0))
        t_at = jnp.sum(jnp.where(sel, tr, 0))
        upd = jnp.logical_and(found == 0, any_)
        B = jnp.where(upd, gi * 16 + 15 - pos, B)
        na = jnp.where(upd, tge_at - t_at, na)
        cnt = jnp.where(upd, t_at, cnt)
        found = jnp.where(any_, 1, found)
        cum = cum + jnp.sum(t)
        return (cum, found, B, na, cnt)
    carry = (cum0, jnp.int32(0), jnp.int32(0), jnp.int32(0), jnp.int32(0))
    _, _, B, na, cnt = lax.fori_loop(0, 64, body, carry)
    return B, na, cnt


def _fill(ref, words, val):
    v = jnp.full((16,), val, jnp.int32)
    def body(i, _):
        ref[pl.ds(i * 16, 16)] = v
        return 0
    lax.fori_loop(0, words // 16, body, 0)


def _collect_thresh(gbuf, ntile, ck, ci, us_lo):
    lane = _lane()
    def body(t, pos):
        for j in range(8):
            us = _u_of(gbuf[t, pl.ds(j * 16, 16)]) ^ INT_MIN32
            m = us >= us_lo
            plsc.store_compressed(ck.at[pl.ds(pos, 16)], us, mask=m)
            plsc.store_compressed(ci.at[pl.ds(pos, 16)],
                                  t * 128 + j * 16 + lane, mask=m)
            pos = pos + jnp.sum(jnp.where(m, 1, 0))
        return pos
    lax.fori_loop(0, ntile, body, jnp.int32(0))


def _collect_exact(gbuf, ntile, ck, ci, ek, ei, pfx):
    lane = _lane()
    us_t = pfx ^ INT_MIN32
    def body(t, carry):
        pos, epos = carry
        for j in range(8):
            us = _u_of(gbuf[t, pl.ds(j * 16, 16)]) ^ INT_MIN32
            idx = t * 128 + j * 16 + lane
            mgt = us > us_t
            plsc.store_compressed(ck.at[pl.ds(pos, 16)], us, mask=mgt)
            plsc.store_compressed(ci.at[pl.ds(pos, 16)], idx, mask=mgt)
            meq = jnp.logical_and(us == us_t, epos < EQW - 16)
            plsc.store_compressed(ek.at[pl.ds(epos, 16)], us, mask=meq)
            plsc.store_compressed(ei.at[pl.ds(epos, 16)], idx, mask=meq)
            pos = pos + jnp.sum(jnp.where(mgt, 1, 0))
            epos = epos + jnp.sum(jnp.where(meq, 1, 0))
        return (pos, epos)
    pos, _ = lax.fori_loop(0, ntile, body, (jnp.int32(0), jnp.int32(0)))
    def copy(i, _):
        ck[pl.ds(pos + i * 16, 16)] = ek[pl.ds(i * 16, 16)]
        ci[pl.ds(pos + i * 16, 16)] = ei[pl.ds(i * 16, 16)]
        return 0
    lax.fori_loop(0, EQW // 16, copy, 0)


def _select100(ck, ci, outv):
    nv = CKW // 16
    lane = _lane()
    def body(j, _):
        mvec = jnp.full((16,), INT_MIN32, jnp.int32)
        for t in range(nv):
            mvec = jnp.maximum(mvec, ck[pl.ds(t * 16, 16)])
        mx = jnp.max(mvec)
        ivec = jnp.full((16,), jnp.int32(0x7FFFFFFF), jnp.int32)
        for t in range(nv):
            k_ = ck[pl.ds(t * 16, 16)]
            i_ = ci[pl.ds(t * 16, 16)]
            ivec = jnp.minimum(ivec, jnp.where(k_ == mx, i_, 0x7FFFFFFF))
        imin = jnp.min(ivec)
        for t in range(nv):
            k_ = ck[pl.ds(t * 16, 16)]
            i_ = ci[pl.ds(t * 16, 16)]
            dead = jnp.logical_and(k_ == mx, i_ == imin)
            ck[pl.ds(t * 16, 16)] = jnp.where(dead, INT_MIN32, k_)
        plsc.store_scatter(outv, [jnp.full((16,), j, jnp.int32)],
                           jnp.full((16,), imin, jnp.int32),
                           mask=lane == 0)
        return 0
    lax.fori_loop(0, TOPK, body, 0)


def _sc_topk(tm_hbm, s2_hbm, out_hbm,
             tmv, gbuf, hist, totals, tids, ck, ci, plist, ek, ei, outv, sem):
    wid = lax.axis_index("s") * 2 + lax.axis_index("c")
    lane = _lane()

    def per_row(qq, _):
        row = wid * ROWS_PER_W + qq
        with jax.named_scope("ph_tm"):
            pltpu.sync_copy(tm_hbm.at[pl.ds(row * NT, NT)], tmv)

        minf = jnp.full((16,), -jnp.inf, jnp.float32)
        def tau_body(i, m):
            x = tmv[pl.ds(i * 16, 16)]
            out = []
            for j in range(7):
                hi = jnp.maximum(m[j], x)
                x = jnp.minimum(m[j], x)
                out.append(hi)
            return tuple(out)
        with jax.named_scope("ph_tau"):
            m7 = lax.fori_loop(0, NTV, tau_body, (minf,) * 7)[6]
            tau = jnp.min(m7)

        def pre_body(i, _):
            tids[pl.ds(i * 16, 16)] = i * 16 + lane
            return 0
        lax.fori_loop(0, NTV, pre_body, 0)
        def surv_body(i, pos):
            msk = tmv[pl.ds(i * 16, 16)] >= tau
            plsc.store_compressed(tids.at[pl.ds(pos, 16)], i * 16 + lane,
                                  mask=msk)
            return pos + jnp.sum(jnp.where(msk, 1, 0))
        with jax.named_scope("ph_surv"):
            ntile = lax.fori_loop(0, NTV, surv_body, jnp.int32(0))
        def abs_body(i, _):
            tids[pl.ds(i * 16, 16)] = tids[pl.ds(i * 16, 16)] + row * NT
            return 0
        lax.fori_loop(0, NTV, abs_body, 0)

        cps = []
        for c in range(7):
            sz = 128 if c < 6 else 32
            cp = pltpu.make_async_copy(
                s2_hbm.at[tids.at[pl.ds(c * 128, sz)]],
                gbuf.at[pl.ds(c * 128, sz)], sem)
            cps.append((cp, c * 128))
        with jax.named_scope("ph_fire"):
            for cp, base in cps:
                pl.when(base < ntile)(lambda cp=cp: cp.start())
        with jax.named_scope("ph_wait"):
            for cp, base in cps:
                pl.when(base < ntile)(lambda cp=cp: cp.wait())

        base = lane * LLEN
        def cbody(t, pl_pos):
            t128 = t * 128
            for j in range(8):
                v = gbuf[t, pl.ds(j * 16, 16)]
                m = v >= tau
                dest = base + jnp.minimum(pl_pos, LLEN - 1)
                plsc.store_scatter(plist, [dest], t128 + j * 16 + lane,
                                   mask=m)
                pl_pos = pl_pos + jnp.where(m, 1, 0)
            return pl_pos
        with jax.named_scope("ph_collect"):
            cnt_vec = lax.fori_loop(0, ntile, cbody,
                                    jnp.zeros((16,), jnp.int32))
        tot = jnp.sum(cnt_vec)
        lmax = jnp.max(cnt_vec)

        def fast(_):
            gpos = jnp.int32(0)
            for l in range(16):
                c_l = jnp.sum(jnp.where(lane == l, cnt_vec, 0))
                for j2 in range(LLEN // 16):
                    def cp(l=l, j2=j2, gpos=gpos):
                        ci[pl.ds(gpos + j2 * 16, 16)] = (
                            plist[pl.ds(l * LLEN + j2 * 16, 16)])
                    pl.when(j2 * 16 < c_l)(cp)
                gpos = gpos + c_l
            for t in range(CKW // 16):
                p0 = ci[pl.ds(t * 16, 16)]
                p = jnp.minimum(jnp.maximum(p0, 0), NT * 128 - 1)
                v = plsc.load_gather(
                    gbuf, [lax.shift_right_logical(p, 7), p & 127])
                b = lax.bitcast_convert_type(v, jnp.int32)
                u = jnp.where(b < 0, ~b, b | INT_MIN32)
                us = u ^ INT_MIN32
                ck[pl.ds(t * 16, 16)] = jnp.where(t * 16 + lane < tot,
                                                  us, INT_MIN32)
            return 0

        def fallback(_):
            _zero_hist(hist)
            _hist_pass(gbuf, ntile, hist, False, 0, 0, 0, 0)
            _reduce_hist(hist, totals)
            B, na, cnt = _scan_bins(totals, jnp.int32(0))

            def refine(carry):
                s, pfx, na, cnt = carry
                s2 = jnp.maximum(s - 10, 0)
                w = s - s2
                _zero_hist(hist)
                _hist_pass(gbuf, ntile, hist, True, s, pfx, s2,
                           lax.shift_left(jnp.int32(1), w) - 1)
                _reduce_hist(hist, totals)
                B2, na2, cnt2 = _scan_bins(totals, na)
                return (s2, lax.shift_left(pfx, w) | B2, na2, cnt2)

            def keep_going(carry):
                s, _, na, cnt = carry
                return jnp.logical_and(na + cnt > CAP, s > 0)

            s, pfx, na, cnt = lax.while_loop(
                keep_going, refine, (jnp.int32(22), B, na, cnt))

            _fill(ck, CKW, INT_MIN32)
            _fill(ci, CKW, 0)

            def coll_fast(_):
                us_lo = lax.shift_left(pfx, s) ^ INT_MIN32
                _collect_thresh(gbuf, ntile, ck, ci, us_lo)
                return 0

            def coll_exact(_):
                _fill(ek, EQW, INT_MIN32)
                _fill(ei, EQW, 0)
                _collect_exact(gbuf, ntile, ck, ci, ek, ei, pfx)
                return 0

            lax.cond(na + cnt <= CAP, coll_fast, coll_exact, 0)
            return 0

        ok = jnp.logical_and(tot <= CAP, lmax <= LLEN)
        with jax.named_scope("ph_merge"):
            lax.cond(ok, fast, fallback, 0)

        with jax.named_scope("ph_sel"):
            _fill(outv, OW, 0)
            _select100(ck, ci, outv)

        with jax.named_scope("ph_map"):
            pass
        for t in range(8):
            p = outv[pl.ds(t * 16, 16)]
            o = lax.shift_right_logical(p, 7)
            tv = plsc.load_gather(tids, [o]) - row * NT
            outv[pl.ds(t * 16, 16)] = tv * 128 + (p & 127)

        pltpu.sync_copy(outv, out_hbm.at[pl.ds(row * OW, OW)])
        return 0

    lax.fori_loop(0, ROWS_PER_W, per_row, 0)


def _topk_indices(s, tm):
    mesh = plsc.VectorSubcoreMesh(core_axis_name="c", subcore_axis_name="s")
    f = functools.partial(
        pl.kernel,
        mesh=mesh,
        compiler_params=pltpu.CompilerParams(needs_layout_passes=False),
        out_type=jax.ShapeDtypeStruct((Q * OW,), jnp.int32),
        scratch_types=[
            pltpu.VMEM((NT,), jnp.float32),
            pltpu.VMEM((NT, 128), jnp.float32),
            pltpu.VMEM((NBIN * 16,), jnp.int32),
            pltpu.VMEM((NBIN,), jnp.int32),
            pltpu.VMEM((NT,), jnp.int32),
            pltpu.VMEM((CKW,), jnp.int32),
            pltpu.VMEM((CKW,), jnp.int32),
            pltpu.VMEM((16 * 64,), jnp.int32),
            pltpu.VMEM((EQW,), jnp.int32),
            pltpu.VMEM((EQW,), jnp.int32),
            pltpu.VMEM((OW,), jnp.int32),
            pltpu.SemaphoreType.DMA,
        ],
    )(_sc_topk)
    return jnp.reshape(f(jnp.reshape(tm, (Q * NT,)), jnp.reshape(s, (Q * NT, 128))), (Q, OW))


def _zs_body(idx_ref, o_ref):
    x = idx_ref[...].astype(jnp.float32)
    lane = lax.broadcasted_iota(jnp.int32, x.shape, 1)
    m = lane < TOPK
    xm = jnp.where(m, x, 0.0)
    mean = jnp.sum(xm, axis=1, keepdims=True) / TOPK
    d = jnp.where(m, x - mean, 0.0)
    var = jnp.sum(d * d, axis=1, keepdims=True) / (TOPK - 1)
    z = d / (jnp.sqrt(var) + 1e-20)
    o_ref[...] = jnp.sum(z, axis=0)


def _zscore_sum(idx):
    return pl.pallas_call(
        _zs_body,
        out_shape=jax.ShapeDtypeStruct((OW,), jnp.float32),
    )(idx)


def kernel(unmatch_entities3, all_candidates3):
    s, tm3 = _scores(unmatch_entities3, all_candidates3)
    tm = jnp.transpose(tm3, (1, 0, 2)).reshape(Q, NT)
    idx = _topk_indices(s, tm)
    out = _zscore_sum(idx)
    return out[:TOPK]


# speedup vs baseline: 1.1399x; 1.1399x over previous
import functools

import jax
import jax.numpy as jnp
from jax import lax
from jax.experimental import pallas as pl
from jax.experimental.pallas import tpu as pltpu
from jax.experimental.pallas import tpu_sc as plsc

Q = 1024
K = 64
N = 100000
NPAD = 102400
TILE = 4096
GRID = NPAD // TILE
NT = NPAD // 128
NTV = NT // 16
TOPK = 100
OW = 128
CAP = 448
CKW = CAP + 16
EQW = 144
NBIN = 1024
LLEN = 64
ROWS_PER_W = Q // 32
INT_MIN32 = -2147483648
PAD_BITS = -8388607


def _mm_body(a_ref, b_ref, s_ref, tm_ref):
    j = pl.program_id(0)
    s = jnp.dot(a_ref[...], b_ref[...], preferred_element_type=jnp.float32)
    col = j * TILE + lax.broadcasted_iota(jnp.int32, s.shape, 1)
    valid = col < N
    s_masked = jnp.where(valid, s, -jnp.inf)
    tm_ref[...] = jnp.max(
        s_masked.reshape(Q, TILE // 128, 128), axis=2)[None]
    pad_val = lax.bitcast_convert_type(jnp.int32(PAD_BITS), jnp.float32)
    s_ref[...] = jnp.where(valid, s, pad_val)


def _scores(unmatch_entities3, all_candidates3):
    b = jnp.pad(all_candidates3, ((0, 0), (0, NPAD - N)))
    return pl.pallas_call(
        _mm_body,
        grid=(GRID,),
        in_specs=[
            pl.BlockSpec((Q, K), lambda j: (0, 0)),
            pl.BlockSpec((K, TILE), lambda j: (0, j)),
        ],
        out_specs=[
            pl.BlockSpec((Q, TILE), lambda j: (0, j)),
            pl.BlockSpec((1, Q, TILE // 128), lambda j: (j, 0, 0)),
        ],
        out_shape=[
            jax.ShapeDtypeStruct((Q, NPAD), jnp.float32),
            jax.ShapeDtypeStruct((GRID, Q, TILE // 128), jnp.float32),
        ],
    )(unmatch_entities3, b)


def _lane():
    return lax.iota(jnp.int32, 16)


def _u_of(v):
    b = lax.bitcast_convert_type(v, jnp.int32)
    return jnp.where(b < 0, ~b, b | INT_MIN32)


def _zero_hist(hist):
    def body(i, _):
        hist[pl.ds(i * 16, 16)] = jnp.zeros((16,), jnp.int32)
        return 0
    lax.fori_loop(0, NBIN, body, 0)


def _hist_pass(gbuf, ntile, hist, masked, s_hi, pfx, s_lo, wmask):
    lane = _lane()
    ones = jnp.ones((16,), jnp.int32)
    def body(t, _):
        for j in range(8):
            u = _u_of(gbuf[t, pl.ds(j * 16, 16)])
            if masked:
                m = lax.shift_right_logical(u, s_hi) == pfx
                bin_ = lax.shift_right_logical(u, s_lo) & wmask
                plsc.addupdate_scatter(hist, [lane * NBIN + bin_], ones,
                                       mask=m)
            else:
                bin_ = lax.shift_right_logical(u, 22)
                plsc.addupdate_scatter(hist, [lane * NBIN + bin_], ones)
        return 0
    lax.fori_loop(0, ntile, body, 0)


def _reduce_hist(hist, totals):
    def body(g, _):
        acc = jnp.zeros((16,), jnp.int32)
        for l in range(16):
            acc = acc + hist[pl.ds(l * NBIN + g * 16, 16)]
        totals[pl.ds(g * 16, 16)] = acc
        return 0
    lax.fori_loop(0, NBIN // 16, body, 0)


def _scan_bins(totals, cum0):
    lane = _lane()
    def body(g, carry):
        cum, found, B, na, cnt = carry
        gi = 63 - g
        t = totals[pl.ds(gi * 16, 16)]
        tr = lax.rev(t, (0,))
        tot_ge = plsc.cumsum(tr) + cum
        posv = jnp.where(tot_ge >= TOPK, lane, 16)
        pos = jnp.min(posv)
        any_ = pos < 16
        sel = lane == pos
        tge_at = jnp.sum(jnp.where(sel, tot_ge, 0))
        t_at = jnp.sum(jnp.where(sel, tr, 0))
        upd = jnp.logical_and(found == 0, any_)
        B = jnp.where(upd, gi * 16 + 15 - pos, B)
        na = jnp.where(upd, tge_at - t_at, na)
        cnt = jnp.where(upd, t_at, cnt)
        found = jnp.where(any_, 1, found)
        cum = cum + jnp.sum(t)
        return (cum, found, B, na, cnt)
    carry = (cum0, jnp.int32(0), jnp.int32(0), jnp.int32(0), jnp.int32(0))
    _, _, B, na, cnt = lax.fori_loop(0, 64, body, carry)
    return B, na, cnt


def _fill(ref, words, val):
    v = jnp.full((16,), val, jnp.int32)
    def body(i, _):
        ref[pl.ds(i * 16, 16)] = v
        return 0
    lax.fori_loop(0, words // 16, body, 0)


def _collect_thresh(gbuf, ntile, ck, ci, us_lo):
    lane = _lane()
    def body(t, pos):
        for j in range(8):
            us = _u_of(gbuf[t, pl.ds(j * 16, 16)]) ^ INT_MIN32
            m = us >= us_lo
            plsc.store_compressed(ck.at[pl.ds(pos, 16)], us, mask=m)
            plsc.store_compressed(ci.at[pl.ds(pos, 16)],
                                  t * 128 + j * 16 + lane, mask=m)
            pos = pos + jnp.sum(jnp.where(m, 1, 0))
        return pos
    lax.fori_loop(0, ntile, body, jnp.int32(0))


def _collect_exact(gbuf, ntile, ck, ci, ek, ei, pfx):
    lane = _lane()
    us_t = pfx ^ INT_MIN32
    def body(t, carry):
        pos, epos = carry
        for j in range(8):
            us = _u_of(gbuf[t, pl.ds(j * 16, 16)]) ^ INT_MIN32
            idx = t * 128 + j * 16 + lane
            mgt = us > us_t
            plsc.store_compressed(ck.at[pl.ds(pos, 16)], us, mask=mgt)
            plsc.store_compressed(ci.at[pl.ds(pos, 16)], idx, mask=mgt)
            meq = jnp.logical_and(us == us_t, epos < EQW - 16)
            plsc.store_compressed(ek.at[pl.ds(epos, 16)], us, mask=meq)
            plsc.store_compressed(ei.at[pl.ds(epos, 16)], idx, mask=meq)
            pos = pos + jnp.sum(jnp.where(mgt, 1, 0))
            epos = epos + jnp.sum(jnp.where(meq, 1, 0))
        return (pos, epos)
    pos, _ = lax.fori_loop(0, ntile, body, (jnp.int32(0), jnp.int32(0)))
    def copy(i, _):
        ck[pl.ds(pos + i * 16, 16)] = ek[pl.ds(i * 16, 16)]
        ci[pl.ds(pos + i * 16, 16)] = ei[pl.ds(i * 16, 16)]
        return 0
    lax.fori_loop(0, EQW // 16, copy, 0)


def _select100(ck, ci, outv):
    nv = CKW // 16
    lane = _lane()
    big = jnp.int32(0x7FFFFFFF)
    smax0 = jnp.full((16,), INT_MIN32, jnp.int32)
    smax1 = jnp.full((16,), INT_MIN32, jnp.int32)
    for t in range(nv):
        m = jnp.max(ck[pl.ds(t * 16, 16)])
        if t < 16:
            smax0 = jnp.where(lane == t, m, smax0)
        else:
            smax1 = jnp.where(lane == (t - 16), m, smax1)

    def body(j, carry):
        smax0, smax1 = carry
        mx = jnp.max(jnp.maximum(smax0, smax1))
        c0 = smax0 == mx
        c1 = smax1 == mx
        pc = (plsc.all_reduce_population_count(c0)
              + plsc.all_reduce_population_count(c1))
        t0 = jnp.min(jnp.where(c0, lane, 64))
        t1 = jnp.min(jnp.where(c1, lane + 16, 64))
        t = jnp.minimum(t0, t1)
        multi = jnp.max(pc) > 1

        def single(_):
            kv = ck[pl.ds(t * 16, 16)]
            iv = ci[pl.ds(t * 16, 16)]
            im = jnp.min(jnp.where(kv == mx, iv, big))
            dead = jnp.logical_and(kv == mx, iv == im)
            kv2 = jnp.where(dead, INT_MIN32, kv)
            ck[pl.ds(t * 16, 16)] = kv2
            nm = jnp.max(kv2)
            u0 = jnp.logical_and(lane == (t & 15), t < 16)
            u1 = jnp.logical_and(lane == (t - 16), t >= 16)
            return (im, jnp.where(u0, nm, smax0), jnp.where(u1, nm, smax1))

        def multi_f(_):
            ivec = jnp.full((16,), big, jnp.int32)
            for tt in range(nv):
                k_ = ck[pl.ds(tt * 16, 16)]
                i_ = ci[pl.ds(tt * 16, 16)]
                ivec = jnp.minimum(ivec, jnp.where(k_ == mx, i_, big))
            im = jnp.min(ivec)
            s0, s1 = smax0, smax1
            for tt in range(nv):
                k_ = ck[pl.ds(tt * 16, 16)]
                i_ = ci[pl.ds(tt * 16, 16)]
                dead = jnp.logical_and(k_ == mx, i_ == im)
                k2 = jnp.where(dead, INT_MIN32, k_)
                ck[pl.ds(tt * 16, 16)] = k2
                nm = jnp.max(k2)
                if tt < 16:
                    s0 = jnp.where(lane == tt, nm, s0)
                else:
                    s1 = jnp.where(lane == (tt - 16), nm, s1)
            return (im, s0, s1)

        im, smax0, smax1 = lax.cond(multi, multi_f, single, 0)
        plsc.store_scatter(outv, [jnp.full((16,), j, jnp.int32)],
                           jnp.full((16,), im, jnp.int32),
                           mask=lane == 0)
        return (smax0, smax1)

    lax.fori_loop(0, TOPK, body, (smax0, smax1))


def _sc_topk(tm_hbm, s2_hbm, out_hbm,
             tmv, gbuf, hist, totals, tids, ck, ci, plist, ek, ei, outv, sem):
    wid = lax.axis_index("s") * 2 + lax.axis_index("c")
    lane = _lane()

    def per_row(qq, _):
        row = wid * ROWS_PER_W + qq
        with jax.named_scope("ph_tm"):
            pltpu.sync_copy(tm_hbm.at[pl.ds(row * NT, NT)], tmv)

        minf = jnp.full((16,), -jnp.inf, jnp.float32)
        def tau_body(i, m):
            x = tmv[pl.ds(i * 16, 16)]
            out = []
            for j in range(7):
                hi = jnp.maximum(m[j], x)
                x = jnp.minimum(m[j], x)
                out.append(hi)
            return tuple(out)
        with jax.named_scope("ph_tau"):
            m7 = lax.fori_loop(0, NTV, tau_body, (minf,) * 7)[6]
            tau = jnp.min(m7)

        def pre_body(i, _):
            tids[pl.ds(i * 16, 16)] = i * 16 + lane
            return 0
        lax.fori_loop(0, NTV, pre_body, 0)
        def surv_body(i, pos):
            msk = tmv[pl.ds(i * 16, 16)] >= tau
            plsc.store_compressed(tids.at[pl.ds(pos, 16)], i * 16 + lane,
                                  mask=msk)
            return pos + jnp.sum(jnp.where(msk, 1, 0))
        with jax.named_scope("ph_surv"):
            ntile = lax.fori_loop(0, NTV, surv_body, jnp.int32(0))
        def abs_body(i, _):
            tids[pl.ds(i * 16, 16)] = tids[pl.ds(i * 16, 16)] + row * NT
            return 0
        lax.fori_loop(0, NTV, abs_body, 0)

        cps = []
        for c in range(7):
            sz = 128 if c < 6 else 32
            cp = pltpu.make_async_copy(
                s2_hbm.at[tids.at[pl.ds(c * 128, sz)]],
                gbuf.at[pl.ds(c * 128, sz)], sem)
            cps.append((cp, c * 128))
        with jax.named_scope("ph_fire"):
            for cp, base in cps:
                pl.when(base < ntile)(lambda cp=cp: cp.start())
        with jax.named_scope("ph_wait"):
            for cp, base in cps:
                pl.when(base < ntile)(lambda cp=cp: cp.wait())

        base = lane * LLEN
        def cbody(t, pl_pos):
            t128 = t * 128
            for j in range(8):
                v = gbuf[t, pl.ds(j * 16, 16)]
                m = v >= tau
                dest = base + jnp.minimum(pl_pos, LLEN - 1)
                plsc.store_scatter(plist, [dest], t128 + j * 16 + lane,
                                   mask=m)
                pl_pos = pl_pos + jnp.where(m, 1, 0)
            return pl_pos
        with jax.named_scope("ph_collect"):
            cnt_vec = lax.fori_loop(0, ntile, cbody,
                                    jnp.zeros((16,), jnp.int32))
        tot = jnp.sum(cnt_vec)
        lmax = jnp.max(cnt_vec)

        def fast(_):
            gpos = jnp.int32(0)
            for l in range(16):
                c_l = jnp.sum(jnp.where(lane == l, cnt_vec, 0))
                for j2 in range(LLEN // 16):
                    def cp(l=l, j2=j2, gpos=gpos):
                        ci[pl.ds(gpos + j2 * 16, 16)] = (
                            plist[pl.ds(l * LLEN + j2 * 16, 16)])
                    pl.when(j2 * 16 < c_l)(cp)
                gpos = gpos + c_l
            for t in range(CKW // 16):
                p0 = ci[pl.ds(t * 16, 16)]
                p = jnp.minimum(jnp.maximum(p0, 0), NT * 128 - 1)
                v = plsc.load_gather(
                    gbuf, [lax.shift_right_logical(p, 7), p & 127])
                b = lax.bitcast_convert_type(v, jnp.int32)
                u = jnp.where(b < 0, ~b, b | INT_MIN32)
                us = u ^ INT_MIN32
                ck[pl.ds(t * 16, 16)] = jnp.where(t * 16 + lane < tot,
                                                  us, INT_MIN32)
            return 0

        def fallback(_):
            _zero_hist(hist)
            _hist_pass(gbuf, ntile, hist, False, 0, 0, 0, 0)
            _reduce_hist(hist, totals)
            B, na, cnt = _scan_bins(totals, jnp.int32(0))

            def refine(carry):
                s, pfx, na, cnt = carry
                s2 = jnp.maximum(s - 10, 0)
                w = s - s2
                _zero_hist(hist)
                _hist_pass(gbuf, ntile, hist, True, s, pfx, s2,
                           lax.shift_left(jnp.int32(1), w) - 1)
                _reduce_hist(hist, totals)
                B2, na2, cnt2 = _scan_bins(totals, na)
                return (s2, lax.shift_left(pfx, w) | B2, na2, cnt2)

            def keep_going(carry):
                s, _, na, cnt = carry
                return jnp.logical_and(na + cnt > CAP, s > 0)

            s, pfx, na, cnt = lax.while_loop(
                keep_going, refine, (jnp.int32(22), B, na, cnt))

            _fill(ck, CKW, INT_MIN32)
            _fill(ci, CKW, 0)

            def coll_fast(_):
                us_lo = lax.shift_left(pfx, s) ^ INT_MIN32
                _collect_thresh(gbuf, ntile, ck, ci, us_lo)
                return 0

            def coll_exact(_):
                _fill(ek, EQW, INT_MIN32)
                _fill(ei, EQW, 0)
                _collect_exact(gbuf, ntile, ck, ci, ek, ei, pfx)
                return 0

            lax.cond(na + cnt <= CAP, coll_fast, coll_exact, 0)
            return 0

        ok = jnp.logical_and(tot <= CAP, lmax <= LLEN)
        with jax.named_scope("ph_merge"):
            lax.cond(ok, fast, fallback, 0)

        with jax.named_scope("ph_sel"):
            _fill(outv, OW, 0)
            _select100(ck, ci, outv)

        with jax.named_scope("ph_map"):
            pass
        for t in range(8):
            p = outv[pl.ds(t * 16, 16)]
            o = lax.shift_right_logical(p, 7)
            tv = plsc.load_gather(tids, [o]) - row * NT
            outv[pl.ds(t * 16, 16)] = tv * 128 + (p & 127)

        pltpu.sync_copy(outv, out_hbm.at[pl.ds(row * OW, OW)])
        return 0

    lax.fori_loop(0, ROWS_PER_W, per_row, 0)


def _topk_indices(s, tm):
    mesh = plsc.VectorSubcoreMesh(core_axis_name="c", subcore_axis_name="s")
    f = functools.partial(
        pl.kernel,
        mesh=mesh,
        compiler_params=pltpu.CompilerParams(needs_layout_passes=False),
        out_type=jax.ShapeDtypeStruct((Q * OW,), jnp.int32),
        scratch_types=[
            pltpu.VMEM((NT,), jnp.float32),
            pltpu.VMEM((NT, 128), jnp.float32),
            pltpu.VMEM((NBIN * 16,), jnp.int32),
            pltpu.VMEM((NBIN,), jnp.int32),
            pltpu.VMEM((NT,), jnp.int32),
            pltpu.VMEM((CKW,), jnp.int32),
            pltpu.VMEM((CKW,), jnp.int32),
            pltpu.VMEM((16 * 64,), jnp.int32),
            pltpu.VMEM((EQW,), jnp.int32),
            pltpu.VMEM((EQW,), jnp.int32),
            pltpu.VMEM((OW,), jnp.int32),
            pltpu.SemaphoreType.DMA,
        ],
    )(_sc_topk)
    return jnp.reshape(f(jnp.reshape(tm, (Q * NT,)), jnp.reshape(s, (Q * NT, 128))), (Q, OW))


def _zs_body(idx_ref, o_ref):
    x = idx_ref[...].astype(jnp.float32)
    lane = lax.broadcasted_iota(jnp.int32, x.shape, 1)
    m = lane < TOPK
    xm = jnp.where(m, x, 0.0)
    mean = jnp.sum(xm, axis=1, keepdims=True) / TOPK
    d = jnp.where(m, x - mean, 0.0)
    var = jnp.sum(d * d, axis=1, keepdims=True) / (TOPK - 1)
    z = d / (jnp.sqrt(var) + 1e-20)
    o_ref[...] = jnp.sum(z, axis=0)


def _zscore_sum(idx):
    return pl.pallas_call(
        _zs_body,
        out_shape=jax.ShapeDtypeStruct((OW,), jnp.float32),
    )(idx)


def kernel(unmatch_entities3, all_candidates3):
    s, tm3 = _scores(unmatch_entities3, all_candidates3)
    tm = jnp.transpose(tm3, (1, 0, 2)).reshape(Q, NT)
    idx = _topk_indices(s, tm)
    out = _zscore_sum(idx)
    return out[:TOPK]
